# baseline (speedup 1.0000x reference)
import jax
import jax.numpy as jnp
from jax import lax
from jax.experimental import pallas as pl
from jax.experimental.pallas import tpu as pltpu
from jax.experimental.pallas import tpu_sc as plsc

T = 2048
H = 2048
I_DIM = 1024
E = 8
R = 602
EPAD = 128
BLK = 256
LOG2_BLK = 8
NB = 24
NSLOT = NB * BLK
NTILES = 16
TPW = T // NTILES
SPW = NSLOT // 32
CPW = T // 32


def _router_body(x_ref, rw_ref, logits_ref, e01_ref, w0_ref):
    xb = x_ref[...]
    rw = rw_ref[...]
    lt = lax.dot_general(xb, rw, (((1,), (1,)), ((), ())),
                         preferred_element_type=jnp.float32)
    logits_ref[...] = lt
    lane = lax.broadcasted_iota(jnp.int32, lt.shape, 1)
    neg = jnp.float32(-jnp.inf)
    lm = jnp.where(lane < E, lt, neg)
    m0 = jnp.max(lm, axis=1, keepdims=True)
    e0 = jnp.min(jnp.where(lm == m0, lane, EPAD), axis=1, keepdims=True)
    lm1 = jnp.where(lane == e0, neg, lm)
    m1 = jnp.max(lm1, axis=1, keepdims=True)
    e1 = jnp.min(jnp.where(lm1 == m1, lane, EPAD), axis=1, keepdims=True)
    w0 = 1.0 / (1.0 + jnp.exp(m1 - m0))
    e01_ref[...] = jnp.broadcast_to(e0 + 16 * e1, lt.shape)
    w0_ref[...] = jnp.broadcast_to(w0, lt.shape)


def _router(x, rw_pad):
    return pl.pallas_call(
        _router_body,
        grid=(T // 256,),
        in_specs=[
            pl.BlockSpec((256, H), lambda i: (i, 0)),
            pl.BlockSpec((EPAD, H), lambda i: (0, 0)),
        ],
        out_specs=[
            pl.BlockSpec((256, EPAD), lambda i: (i, 0)),
            pl.BlockSpec((256, EPAD), lambda i: (i, 0)),
            pl.BlockSpec((256, EPAD), lambda i: (i, 0)),
        ],
        out_shape=[
            jax.ShapeDtypeStruct((T, EPAD), jnp.float32),
            jax.ShapeDtypeStruct((T, EPAD), jnp.int32),
            jax.ShapeDtypeStruct((T, EPAD), jnp.float32),
        ],
    )(x, rw_pad)


def _base_body(x_ref, wg_ref, wu_ref, bg_ref, bu_ref):
    xb = x_ref[...]
    cdim = (((1,), (1,)), ((), ()))
    bg_ref[...] = lax.dot_general(xb, wg_ref[...], cdim,
                                  preferred_element_type=jnp.float32)
    bu_ref[...] = lax.dot_general(xb, wu_ref[...], cdim,
                                  preferred_element_type=jnp.float32)


def _base(x, wg, wu):
    return pl.pallas_call(
        _base_body,
        grid=(T // 256,),
        in_specs=[
            pl.BlockSpec((256, H), lambda i: (i, 0)),
            pl.BlockSpec((I_DIM, H), lambda i: (0, 0)),
            pl.BlockSpec((I_DIM, H), lambda i: (0, 0)),
        ],
        out_specs=[
            pl.BlockSpec((256, I_DIM), lambda i: (i, 0)),
            pl.BlockSpec((256, I_DIM), lambda i: (i, 0)),
        ],
        out_shape=[
            jax.ShapeDtypeStruct((T, I_DIM), jnp.float32),
            jax.ShapeDtypeStruct((T, I_DIM), jnp.float32),
        ],
    )(x, wg, wu)


def _splat(v, p):
    return jnp.take(v, jnp.full((16,), p, jnp.int32))


def _route_body(iota_hbm, e01_hbm, w0_hbm,
                stok_hbm, sw_hbm, blke_hbm, blkv_hbm, pos_hbm, cnt_hbm,
                iov, e01v, w0v, ev, w1v, cnt, allcnt, zeri, zerf,
                tok, pos0, pos1, bvm, bvv):
    w = lax.axis_index("s")
    i32 = jnp.int32
    pltpu.sync_copy(iota_hbm, iov)
    iot = iov[...]
    pltpu.sync_copy(e01_hbm.at[pl.ds(w * TPW, TPW)], e01v)
    pltpu.sync_copy(w0_hbm.at[pl.ds(w * TPW, TPW)], w0v)
    cntv = jnp.zeros((16,), i32)
    for g in range(TPW // 16):
        pk = e01v[pl.ds(g * 16, 16)]
        e0 = pk & 15
        e1 = pk >> 4
        ev[pl.ds(g * 16, 16)] = e0
        ev[pl.ds(TPW + g * 16, 16)] = e1
        w1v[pl.ds(g * 16, 16)] = 1.0 - w0v[pl.ds(g * 16, 16)]
        tok[pl.ds(g * 16, 16)] = w * TPW + g * 16 + iot
        for p in range(16):
            cntv = cntv + jnp.where(iot == jnp.take(e0, e0 * 0 + p), 1, 0)
            cntv = cntv + jnp.where(iot == jnp.take(e1, e1 * 0 + p), 1, 0)
    cnt[...] = cntv
    nz = NSLOT // NTILES
    for j in range(nz // 16):
        zeri[pl.ds(j * 16, 16)] = jnp.zeros((16,), i32)
        zerf[pl.ds(j * 16, 16)] = jnp.zeros((16,), jnp.float32)
    pltpu.sync_copy(zeri, stok_hbm.at[pl.ds(w * nz, nz)])
    pltpu.sync_copy(zerf, sw_hbm.at[pl.ds(w * nz, nz)])
    pltpu.sync_copy(cnt, cnt_hbm.at[pl.ds(w * 16, 16)])
    plsc.subcore_barrier()
    pltpu.sync_copy(cnt_hbm, allcnt)
    tot = jnp.zeros((16,), i32)
    pre = jnp.zeros((16,), i32)
    for j in range(NTILES):
        row = allcnt[pl.ds(j * 16, 16)]
        tot = tot + row
        pre = pre + row * (j < w).astype(i32)
    pc = ((tot + (BLK - 1)) >> LOG2_BLK) << LOG2_BLK
    basev = jnp.zeros((16,), i32)
    accv = pc * 0
    nbc = []
    for e in range(E):
        basev = basev + jnp.where(iot == e, accv, 0)
        accv = accv + jnp.take(pc, pc * 0 + e)
        nbc.append(accv >> LOG2_BLK)
    start = basev + pre

    @pl.when(w == 0)
    def _write_block_tables():
        for half in range(2):
            b = iot + half * 16
            be = jnp.zeros((16,), i32)
            for e in range(E - 1):
                be = be + jnp.where(b >= nbc[e], 1, 0)
            valid = jnp.where(b < nbc[E - 1], 1, 0)
            be = be * valid + (E - 1) * (1 - valid)
            bvm[pl.ds(half * 16, 16)] = be
            bvv[pl.ds(half * 16, 16)] = valid
        pltpu.sync_copy(bvm, blke_hbm)
        pltpu.sync_copy(bvv, blkv_hbm)

    off = start
    gt = [jnp.where(iot > p, 1, 0) for p in range(16)]
    for k in range(2):
        dst = pos0 if k == 0 else pos1
        for g in range(TPW // 16):
            gbase = k * TPW + g * 16
            ids = ev[pl.ds(gbase, 16)]
            rank = jnp.zeros((16,), i32)
            hist = jnp.zeros((16,), i32)
            for p in range(16):
                sp = jnp.take(ids, ids * 0 + p)
                rank = rank + jnp.where(ids == sp, gt[p], 0)
                hist = hist + jnp.where(iot == sp, 1, 0)
            dst[pl.ds(g * 16, 16)] = jnp.take(off, ids) + rank
            off = off + hist
    pltpu.sync_copy(tok, stok_hbm.at[pos0])
    pltpu.sync_copy(tok, stok_hbm.at[pos1])
    pltpu.sync_copy(w0v, sw_hbm.at[pos0])
    pltpu.sync_copy(w1v, sw_hbm.at[pos1])
    pltpu.sync_copy(pos0, pos_hbm.at[pl.ds(w * TPW, TPW)])
    pltpu.sync_copy(pos1, pos_hbm.at[pl.ds(T + w * TPW, TPW)])


def _route(iota16, e01, w0):
    mesh = plsc.VectorSubcoreMesh(core_axis_name="c", subcore_axis_name="s",
                                  num_cores=1, num_subcores=NTILES)
    f = pl.kernel(
        _route_body,
        out_type=[
            jax.ShapeDtypeStruct((NSLOT,), jnp.int32),
            jax.ShapeDtypeStruct((NSLOT,), jnp.float32),
            jax.ShapeDtypeStruct((32,), jnp.int32),
            jax.ShapeDtypeStruct((32,), jnp.int32),
            jax.ShapeDtypeStruct((2 * T,), jnp.int32),
            jax.ShapeDtypeStruct((NTILES * 16,), jnp.int32),
        ],
        mesh=mesh,
        scratch_types=[
            pltpu.VMEM((16,), jnp.int32),
            pltpu.VMEM((TPW,), jnp.int32),
            pltpu.VMEM((TPW,), jnp.float32),
            pltpu.VMEM((2 * TPW,), jnp.int32),
            pltpu.VMEM((TPW,), jnp.float32),
            pltpu.VMEM((16,), jnp.int32),
            pltpu.VMEM((NTILES * 16,), jnp.int32),
            pltpu.VMEM((NSLOT // NTILES,), jnp.int32),
            pltpu.VMEM((NSLOT // NTILES,), jnp.float32),
            pltpu.VMEM((TPW,), jnp.int32),
            pltpu.VMEM((TPW,), jnp.int32),
            pltpu.VMEM((TPW,), jnp.int32),
            pltpu.VMEM((32,), jnp.int32),
            pltpu.VMEM((32,), jnp.int32),
        ],
    )
    return f(iota16, e01, w0)


def _gather_body(stok_hbm, x_hbm, bg_hbm, bu_hbm, xg_hbm, gg_hbm, gu_hbm,
                 idx0, idx1, x0, g0, u0, x1, g1, u1, sem0, sem1):
    wid = lax.axis_index("s") * 2 + lax.axis_index("c")
    base = wid * SPW
    nc = SPW // 16
    bufs = [(idx0, x0, g0, u0, sem0), (idx1, x1, g1, u1, sem1)]

    def issue(c, bi):
        idxr, xr, gr, ur, sem = bufs[bi]
        b = base + c * 16
        pltpu.sync_copy(stok_hbm.at[pl.ds(b, 16)], idxr)
        return (pltpu.async_copy(x_hbm.at[idxr], xr, sem),
                pltpu.async_copy(bg_hbm.at[idxr], gr, sem),
                pltpu.async_copy(bu_hbm.at[idxr], ur, sem))

    handles = [issue(0, 0), None]
    for c in range(nc):
        bi = c % 2
        if c + 1 < nc:
            handles[1 - bi] = issue(c + 1, 1 - bi)
        for h in handles[bi]:
            h.wait()
        _, xr, gr, ur, _ = bufs[bi]
        b = base + c * 16
        pltpu.sync_copy(xr, xg_hbm.at[pl.ds(b, 16)])
        pltpu.sync_copy(gr, gg_hbm.at[pl.ds(b, 16)])
        pltpu.sync_copy(ur, gu_hbm.at[pl.ds(b, 16)])


def _gather(stok, x_bf, baseg, baseu):
    mesh = plsc.VectorSubcoreMesh(core_axis_name="c", subcore_axis_name="s",
                                  num_cores=2, num_subcores=NTILES)
    f = pl.kernel(
        _gather_body,
        out_type=[
            jax.ShapeDtypeStruct((NSLOT, H // 2), jnp.int32),
            jax.ShapeDtypeStruct((NSLOT, I_DIM), jnp.float32),
            jax.ShapeDtypeStruct((NSLOT, I_DIM), jnp.float32),
        ],
        mesh=mesh,
        scratch_types=[
            pltpu.VMEM((16,), jnp.int32),
            pltpu.VMEM((16,), jnp.int32),
            pltpu.VMEM((16, H // 2), jnp.int32),
            pltpu.VMEM((16, I_DIM), jnp.float32),
            pltpu.VMEM((16, I_DIM), jnp.float32),
            pltpu.VMEM((16, H // 2), jnp.int32),
            pltpu.VMEM((16, I_DIM), jnp.float32),
            pltpu.VMEM((16, I_DIM), jnp.float32),
            pltpu.SemaphoreType.DMA,
            pltpu.SemaphoreType.DMA,
        ],
    )
    return f(stok, x_bf, baseg, baseu)


def _gu_body(be_ref, bv_ref, xg_ref, gg_ref, gu_ref,
             ag_ref, bg_ref, au_ref, bu_ref, mid_ref):
    i = pl.program_id(0)

    @pl.when(bv_ref[i] == 1)
    def _():
        xb = xg_ref[...]
        cdim = (((1,), (1,)), ((), ()))
        f32 = jnp.float32
        g = gg_ref[...] + lax.dot_general(
            lax.dot_general(xb, ag_ref[0], cdim, preferred_element_type=f32),
            bg_ref[0], cdim, preferred_element_type=f32)
        u = gu_ref[...] + lax.dot_general(
            lax.dot_general(xb, au_ref[0], cdim, preferred_element_type=f32),
            bu_ref[0], cdim, preferred_element_type=f32)
        mid_ref[...] = (g * jax.nn.sigmoid(g) * u).astype(jnp.bfloat16)


def _gu(blke, blkv, xg, gg, gu, ag, bg, au, bu):
    grid_spec = pltpu.PrefetchScalarGridSpec(
        num_scalar_prefetch=2,
        grid=(NB,),
        in_specs=[
            pl.BlockSpec((BLK, H), lambda i, be, bv: (i, 0)),
            pl.BlockSpec((BLK, I_DIM), lambda i, be, bv: (i, 0)),
            pl.BlockSpec((BLK, I_DIM), lambda i, be, bv: (i, 0)),
            pl.BlockSpec((1, R, H), lambda i, be, bv: (be[i], 0, 0)),
            pl.BlockSpec((1, I_DIM, R), lambda i, be, bv: (be[i], 0, 0)),
            pl.BlockSpec((1, R, H), lambda i, be, bv: (be[i], 0, 0)),
            pl.BlockSpec((1, I_DIM, R), lambda i, be, bv: (be[i], 0, 0)),
        ],
        out_specs=pl.BlockSpec((BLK, I_DIM), lambda i, be, bv: (i, 0)),
    )
    return pl.pallas_call(
        _gu_body,
        grid_spec=grid_spec,
        out_shape=jax.ShapeDtypeStruct((NSLOT, I_DIM), jnp.bfloat16),
    )(blke, blkv, xg, gg, gu, ag, bg, au, bu)


def _down_body(be_ref, bv_ref, mid_ref, wd_ref, ad_ref, bd_ref, sw_ref, out_ref):
    i = pl.program_id(0)

    @pl.when(bv_ref[i] == 1)
    def _():
        mid = mid_ref[...]
        cdim = (((1,), (1,)), ((), ()))
        f32 = jnp.float32
        d = (lax.dot_general(mid, wd_ref[...], cdim, preferred_element_type=f32)
             + lax.dot_general(
                 lax.dot_general(mid, ad_ref[0], cdim, preferred_element_type=f32),
                 bd_ref[0], cdim, preferred_element_type=f32))
        out_ref[...] = d * sw_ref[0, 0, :][:, None]


def _down(blke, blkv, mid, wd, ad, bd, sw3):
    grid_spec = pltpu.PrefetchScalarGridSpec(
        num_scalar_prefetch=2,
        grid=(NB,),
        in_specs=[
            pl.BlockSpec((BLK, I_DIM), lambda i, be, bv: (i, 0)),
            pl.BlockSpec((H, I_DIM), lambda i, be, bv: (0, 0)),
            pl.BlockSpec((1, R, I_DIM), lambda i, be, bv: (be[i], 0, 0)),
            pl.BlockSpec((1, H, R), lambda i, be, bv: (be[i], 0, 0)),
            pl.BlockSpec((1, 1, BLK), lambda i, be, bv: (i, 0, 0)),
        ],
        out_specs=pl.BlockSpec((BLK, H), lambda i, be, bv: (i, 0)),
    )
    return pl.pallas_call(
        _down_body,
        grid_spec=grid_spec,
        out_shape=jax.ShapeDtypeStruct((NSLOT, H), jnp.float32),
    )(blke, blkv, mid, wd, ad, bd, sw3)


def _combine_body(pos_hbm, op_hbm, out_hbm, i0, i1, r0, r1, sem0, sem1):
    wid = lax.axis_index("s") * 2 + lax.axis_index("c")
    for c in range(CPW // 16):
        tb = wid * CPW + c * 16
        pltpu.sync_copy(pos_hbm.at[pl.ds(tb, 16)], i0)
        pltpu.sync_copy(pos_hbm.at[pl.ds(T + tb, 16)], i1)
        pltpu.async_copy(op_hbm.at[i0], r0, sem0).wait()
        pltpu.async_copy(op_hbm.at[i1], r1, sem1).wait()
        for row in range(16):
            def add_row(j, _, row=row):
                cc = j * 16
                r0[row, pl.ds(cc, 16)] = (r0[row, pl.ds(cc, 16)]
                                          + r1[row, pl.ds(cc, 16)])
                return 0
            lax.fori_loop(0, H // 16, add_row, 0)
        pltpu.sync_copy(r0, out_hbm.at[pl.ds(tb, 16)])


def _combine(pos, outp):
    mesh = plsc.VectorSubcoreMesh(core_axis_name="c", subcore_axis_name="s",
                                  num_cores=2, num_subcores=NTILES)
    f = pl.kernel(
        _combine_body,
        out_type=jax.ShapeDtypeStruct((T, H), jnp.float32),
        mesh=mesh,
        scratch_types=[
            pltpu.VMEM((16,), jnp.int32),
            pltpu.VMEM((16,), jnp.int32),
            pltpu.VMEM((16, H), jnp.float32),
            pltpu.VMEM((16, H), jnp.float32),
            pltpu.SemaphoreType.DMA,
            pltpu.SemaphoreType.DMA,
        ],
    )
    return f(pos, outp)


def kernel(hidden_states, router_w, base_gate_w, base_up_w, base_down_w,
           lora_gate_a, lora_gate_b, lora_up_a, lora_up_b,
           lora_down_a, lora_down_b):
    bf16 = jnp.bfloat16
    x = hidden_states.reshape(T, H)
    x_bf = x.astype(bf16)
    rw_pad = jnp.zeros((EPAD, H), bf16).at[:E].set(router_w.astype(bf16))
    logits_full, e01_full, w0_full = _router(x_bf, rw_pad)
    router_logits = logits_full[:, :E]
    baseg, baseu = _base(x_bf, base_gate_w.astype(bf16), base_up_w.astype(bf16))
    iota16 = jnp.arange(16, dtype=jnp.int32)
    stok, sw, blke, blkv, pos, _ = _route(iota16, e01_full[:, 0], w0_full[:, 0])
    x_pack = lax.bitcast_convert_type(x_bf.reshape(T, H // 2, 2), jnp.int32)
    xg32, gg, gu = _gather(stok, x_pack, baseg, baseu)
    xg = lax.bitcast_convert_type(xg32, bf16).reshape(NSLOT, H)
    mid = _gu(blke, blkv, xg, gg, gu,
              lora_gate_a.astype(bf16), lora_gate_b.astype(bf16),
              lora_up_a.astype(bf16), lora_up_b.astype(bf16))
    outp = _down(blke, blkv, mid, base_down_w.astype(bf16),
                 lora_down_a.astype(bf16), lora_down_b.astype(bf16),
                 sw.reshape(NB, 1, BLK))
    final = _combine(pos, outp)
    return final.reshape(1, T, H), router_logits


# speedup vs baseline: 1.0242x; 1.0242x over previous
import jax
import jax.numpy as jnp
from jax import lax
from jax.experimental import pallas as pl
from jax.experimental.pallas import tpu as pltpu
from jax.experimental.pallas import tpu_sc as plsc

T = 2048
H = 2048
I_DIM = 1024
E = 8
R = 602
EPAD = 128
BLK = 256
LOG2_BLK = 8
NB = 24
NSLOT = NB * BLK
NTILES = 16
TPW = T // NTILES
SPW = NSLOT // 32
CPW = T // 32


def _router_body(x_ref, rw_ref, logits_ref, e01_ref, w0_ref):
    xb = x_ref[...]
    rw = rw_ref[...]
    lt = lax.dot_general(xb, rw, (((1,), (1,)), ((), ())),
                         preferred_element_type=jnp.float32)
    logits_ref[...] = lt
    lane = lax.broadcasted_iota(jnp.int32, lt.shape, 1)
    neg = jnp.float32(-jnp.inf)
    lm = jnp.where(lane < E, lt, neg)
    m0 = jnp.max(lm, axis=1, keepdims=True)
    e0 = jnp.min(jnp.where(lm == m0, lane, EPAD), axis=1, keepdims=True)
    lm1 = jnp.where(lane == e0, neg, lm)
    m1 = jnp.max(lm1, axis=1, keepdims=True)
    e1 = jnp.min(jnp.where(lm1 == m1, lane, EPAD), axis=1, keepdims=True)
    w0 = 1.0 / (1.0 + jnp.exp(m1 - m0))
    e01_ref[...] = jnp.broadcast_to(e0 + 16 * e1, lt.shape)
    w0_ref[...] = jnp.broadcast_to(w0, lt.shape)


def _router(x, rw_pad):
    return pl.pallas_call(
        _router_body,
        grid=(T // 256,),
        in_specs=[
            pl.BlockSpec((256, H), lambda i: (i, 0)),
            pl.BlockSpec((EPAD, H), lambda i: (0, 0)),
        ],
        out_specs=[
            pl.BlockSpec((256, EPAD), lambda i: (i, 0)),
            pl.BlockSpec((256, EPAD), lambda i: (i, 0)),
            pl.BlockSpec((256, EPAD), lambda i: (i, 0)),
        ],
        out_shape=[
            jax.ShapeDtypeStruct((T, EPAD), jnp.float32),
            jax.ShapeDtypeStruct((T, EPAD), jnp.int32),
            jax.ShapeDtypeStruct((T, EPAD), jnp.float32),
        ],
    )(x, rw_pad)


def _base_body(x_ref, wg_ref, wu_ref, bg_ref, bu_ref):
    xb = x_ref[...]
    cdim = (((1,), (1,)), ((), ()))
    bg_ref[...] = lax.dot_general(xb, wg_ref[...], cdim,
                                  preferred_element_type=jnp.float32)
    bu_ref[...] = lax.dot_general(xb, wu_ref[...], cdim,
                                  preferred_element_type=jnp.float32)


def _base(x, wg, wu):
    return pl.pallas_call(
        _base_body,
        grid=(T // 256,),
        in_specs=[
            pl.BlockSpec((256, H), lambda i: (i, 0)),
            pl.BlockSpec((I_DIM, H), lambda i: (0, 0)),
            pl.BlockSpec((I_DIM, H), lambda i: (0, 0)),
        ],
        out_specs=[
            pl.BlockSpec((256, I_DIM), lambda i: (i, 0)),
            pl.BlockSpec((256, I_DIM), lambda i: (i, 0)),
        ],
        out_shape=[
            jax.ShapeDtypeStruct((T, I_DIM), jnp.float32),
            jax.ShapeDtypeStruct((T, I_DIM), jnp.float32),
        ],
    )(x, wg, wu)


def _splat(v, p):
    return jnp.take(v, jnp.full((16,), p, jnp.int32))


def _route_body(iota_hbm, e01_hbm, w0_hbm,
                stok_hbm, sw_hbm, blke_hbm, blkv_hbm, pos_hbm, cnt_hbm,
                iov, e01v, w0v, ev, w1v, cnt, allcnt, zeri, zerf,
                tok, pos0, pos1, bvm, bvv):
    w = lax.axis_index("s")
    i32 = jnp.int32
    pltpu.sync_copy(iota_hbm, iov)
    iot = iov[...]
    pltpu.sync_copy(e01_hbm.at[pl.ds(w * TPW, TPW)], e01v)
    pltpu.sync_copy(w0_hbm.at[pl.ds(w * TPW, TPW)], w0v)
    cntv = jnp.zeros((16,), i32)
    for g in range(TPW // 16):
        pk = e01v[pl.ds(g * 16, 16)]
        e0 = pk & 15
        e1 = pk >> 4
        ev[pl.ds(g * 16, 16)] = e0
        ev[pl.ds(TPW + g * 16, 16)] = e1
        w1v[pl.ds(g * 16, 16)] = 1.0 - w0v[pl.ds(g * 16, 16)]
        tok[pl.ds(g * 16, 16)] = w * TPW + g * 16 + iot
        for p in range(16):
            cntv = cntv + jnp.where(iot == jnp.take(e0, e0 * 0 + p), 1, 0)
            cntv = cntv + jnp.where(iot == jnp.take(e1, e1 * 0 + p), 1, 0)
    cnt[...] = cntv
    nz = NSLOT // NTILES
    for j in range(nz // 16):
        zeri[pl.ds(j * 16, 16)] = jnp.zeros((16,), i32)
        zerf[pl.ds(j * 16, 16)] = jnp.zeros((16,), jnp.float32)
    pltpu.sync_copy(zeri, stok_hbm.at[pl.ds(w * nz, nz)])
    pltpu.sync_copy(zerf, sw_hbm.at[pl.ds(w * nz, nz)])
    pltpu.sync_copy(cnt, cnt_hbm.at[pl.ds(w * 16, 16)])
    plsc.subcore_barrier()
    pltpu.sync_copy(cnt_hbm, allcnt)
    tot = jnp.zeros((16,), i32)
    pre = jnp.zeros((16,), i32)
    for j in range(NTILES):
        row = allcnt[pl.ds(j * 16, 16)]
        tot = tot + row
        pre = pre + row * (j < w).astype(i32)
    pc = ((tot + (BLK - 1)) >> LOG2_BLK) << LOG2_BLK
    basev = jnp.zeros((16,), i32)
    accv = pc * 0
    nbc = []
    for e in range(E):
        basev = basev + jnp.where(iot == e, accv, 0)
        accv = accv + jnp.take(pc, pc * 0 + e)
        nbc.append(accv >> LOG2_BLK)
    start = basev + pre

    @pl.when(w == 0)
    def _write_block_tables():
        for half in range(2):
            b = iot + half * 16
            be = jnp.zeros((16,), i32)
            for e in range(E - 1):
                be = be + jnp.where(b >= nbc[e], 1, 0)
            valid = jnp.where(b < nbc[E - 1], 1, 0)
            be = be * valid + (E - 1) * (1 - valid)
            bvm[pl.ds(half * 16, 16)] = be
            bvv[pl.ds(half * 16, 16)] = valid
        pltpu.sync_copy(bvm, blke_hbm)
        pltpu.sync_copy(bvv, blkv_hbm)

    off = start
    gt = [jnp.where(iot > p, 1, 0) for p in range(16)]
    for k in range(2):
        dst = pos0 if k == 0 else pos1
        for g in range(TPW // 16):
            gbase = k * TPW + g * 16
            ids = ev[pl.ds(gbase, 16)]
            rank = jnp.zeros((16,), i32)
            hist = jnp.zeros((16,), i32)
            for p in range(16):
                sp = jnp.take(ids, ids * 0 + p)
                rank = rank + jnp.where(ids == sp, gt[p], 0)
                hist = hist + jnp.where(iot == sp, 1, 0)
            dst[pl.ds(g * 16, 16)] = jnp.take(off, ids) + rank
            off = off + hist
    pltpu.sync_copy(tok, stok_hbm.at[pos0])
    pltpu.sync_copy(tok, stok_hbm.at[pos1])
    pltpu.sync_copy(w0v, sw_hbm.at[pos0])
    pltpu.sync_copy(w1v, sw_hbm.at[pos1])
    pltpu.sync_copy(pos0, pos_hbm.at[pl.ds(w * TPW, TPW)])
    pltpu.sync_copy(pos1, pos_hbm.at[pl.ds(T + w * TPW, TPW)])


def _route(iota16, e01, w0):
    mesh = plsc.VectorSubcoreMesh(core_axis_name="c", subcore_axis_name="s",
                                  num_cores=1, num_subcores=NTILES)
    f = pl.kernel(
        _route_body,
        out_type=[
            jax.ShapeDtypeStruct((NSLOT,), jnp.int32),
            jax.ShapeDtypeStruct((NSLOT,), jnp.float32),
            jax.ShapeDtypeStruct((32,), jnp.int32),
            jax.ShapeDtypeStruct((32,), jnp.int32),
            jax.ShapeDtypeStruct((2 * T,), jnp.int32),
            jax.ShapeDtypeStruct((NTILES * 16,), jnp.int32),
        ],
        mesh=mesh,
        scratch_types=[
            pltpu.VMEM((16,), jnp.int32),
            pltpu.VMEM((TPW,), jnp.int32),
            pltpu.VMEM((TPW,), jnp.float32),
            pltpu.VMEM((2 * TPW,), jnp.int32),
            pltpu.VMEM((TPW,), jnp.float32),
            pltpu.VMEM((16,), jnp.int32),
            pltpu.VMEM((NTILES * 16,), jnp.int32),
            pltpu.VMEM((NSLOT // NTILES,), jnp.int32),
            pltpu.VMEM((NSLOT // NTILES,), jnp.float32),
            pltpu.VMEM((TPW,), jnp.int32),
            pltpu.VMEM((TPW,), jnp.int32),
            pltpu.VMEM((TPW,), jnp.int32),
            pltpu.VMEM((32,), jnp.int32),
            pltpu.VMEM((32,), jnp.int32),
        ],
    )
    return f(iota16, e01, w0)


def _gather_body(stok_hbm, x_hbm, bg_hbm, bu_hbm, xg_hbm, gg_hbm, gu_hbm,
                 idx0, idx1, x0, g0, u0, x1, g1, u1, sem0, sem1):
    wid = lax.axis_index("s") * 2 + lax.axis_index("c")
    base = wid * SPW
    nc = SPW // 16
    bufs = [(idx0, x0, g0, u0, sem0), (idx1, x1, g1, u1, sem1)]

    def issue(c, bi):
        idxr, xr, gr, ur, sem = bufs[bi]
        b = base + c * 16
        pltpu.sync_copy(stok_hbm.at[pl.ds(b, 16)], idxr)
        return (pltpu.async_copy(x_hbm.at[idxr], xr, sem),
                pltpu.async_copy(bg_hbm.at[idxr], gr, sem),
                pltpu.async_copy(bu_hbm.at[idxr], ur, sem))

    handles = [issue(0, 0), None]
    for c in range(nc):
        bi = c % 2
        if c + 1 < nc:
            handles[1 - bi] = issue(c + 1, 1 - bi)
        for h in handles[bi]:
            h.wait()
        _, xr, gr, ur, _ = bufs[bi]
        b = base + c * 16
        pltpu.sync_copy(xr, xg_hbm.at[pl.ds(b, 16)])
        pltpu.sync_copy(gr, gg_hbm.at[pl.ds(b, 16)])
        pltpu.sync_copy(ur, gu_hbm.at[pl.ds(b, 16)])


def _gather(stok, x_bf, baseg, baseu):
    mesh = plsc.VectorSubcoreMesh(core_axis_name="c", subcore_axis_name="s",
                                  num_cores=2, num_subcores=NTILES)
    f = pl.kernel(
        _gather_body,
        out_type=[
            jax.ShapeDtypeStruct((NSLOT, H // 2), jnp.int32),
            jax.ShapeDtypeStruct((NSLOT, I_DIM), jnp.float32),
            jax.ShapeDtypeStruct((NSLOT, I_DIM), jnp.float32),
        ],
        mesh=mesh,
        scratch_types=[
            pltpu.VMEM((16,), jnp.int32),
            pltpu.VMEM((16,), jnp.int32),
            pltpu.VMEM((16, H // 2), jnp.int32),
            pltpu.VMEM((16, I_DIM), jnp.float32),
            pltpu.VMEM((16, I_DIM), jnp.float32),
            pltpu.VMEM((16, H // 2), jnp.int32),
            pltpu.VMEM((16, I_DIM), jnp.float32),
            pltpu.VMEM((16, I_DIM), jnp.float32),
            pltpu.SemaphoreType.DMA,
            pltpu.SemaphoreType.DMA,
        ],
    )
    return f(stok, x_bf, baseg, baseu)


def _gu_body(be_ref, bv_ref, xg_ref, gg_ref, gu_ref,
             ag_ref, bg_ref, au_ref, bu_ref, mid_ref):
    i = pl.program_id(0)

    @pl.when(bv_ref[i] == 1)
    def _():
        xb = xg_ref[...]
        cdim = (((1,), (1,)), ((), ()))
        f32 = jnp.float32
        g = gg_ref[...] + lax.dot_general(
            lax.dot_general(xb, ag_ref[0], cdim, preferred_element_type=f32),
            bg_ref[0], cdim, preferred_element_type=f32)
        u = gu_ref[...] + lax.dot_general(
            lax.dot_general(xb, au_ref[0], cdim, preferred_element_type=f32),
            bu_ref[0], cdim, preferred_element_type=f32)
        mid_ref[...] = g * jax.nn.sigmoid(g) * u


def _gu(blke, blkv, xg, gg, gu, ag, bg, au, bu):
    grid_spec = pltpu.PrefetchScalarGridSpec(
        num_scalar_prefetch=2,
        grid=(NB,),
        in_specs=[
            pl.BlockSpec((BLK, H), lambda i, be, bv: (i, 0)),
            pl.BlockSpec((BLK, I_DIM), lambda i, be, bv: (i, 0)),
            pl.BlockSpec((BLK, I_DIM), lambda i, be, bv: (i, 0)),
            pl.BlockSpec((1, R, H), lambda i, be, bv: (be[i], 0, 0)),
            pl.BlockSpec((1, I_DIM, R), lambda i, be, bv: (be[i], 0, 0)),
            pl.BlockSpec((1, R, H), lambda i, be, bv: (be[i], 0, 0)),
            pl.BlockSpec((1, I_DIM, R), lambda i, be, bv: (be[i], 0, 0)),
        ],
        out_specs=pl.BlockSpec((BLK, I_DIM), lambda i, be, bv: (i, 0)),
    )
    return pl.pallas_call(
        _gu_body,
        grid_spec=grid_spec,
        out_shape=jax.ShapeDtypeStruct((NSLOT, I_DIM), jnp.float32),
    )(blke, blkv, xg, gg, gu, ag, bg, au, bu)


def _down_body(be_ref, bv_ref, mid_ref, wd_ref, ad_ref, bd_ref, sw_ref, out_ref):
    i = pl.program_id(0)

    @pl.when(bv_ref[i] == 1)
    def _():
        mid = mid_ref[...]
        cdim = (((1,), (1,)), ((), ()))
        f32 = jnp.float32
        d = (lax.dot_general(mid, wd_ref[...], cdim, preferred_element_type=f32)
             + lax.dot_general(
                 lax.dot_general(mid, ad_ref[0], cdim, preferred_element_type=f32),
                 bd_ref[0], cdim, preferred_element_type=f32))
        out_ref[...] = d * sw_ref[0, 0, :][:, None]


def _down(blke, blkv, mid, wd, ad, bd, sw3):
    grid_spec = pltpu.PrefetchScalarGridSpec(
        num_scalar_prefetch=2,
        grid=(NB,),
        in_specs=[
            pl.BlockSpec((BLK, I_DIM), lambda i, be, bv: (i, 0)),
            pl.BlockSpec((H, I_DIM), lambda i, be, bv: (0, 0)),
            pl.BlockSpec((1, R, I_DIM), lambda i, be, bv: (be[i], 0, 0)),
            pl.BlockSpec((1, H, R), lambda i, be, bv: (be[i], 0, 0)),
            pl.BlockSpec((1, 1, BLK), lambda i, be, bv: (i, 0, 0)),
        ],
        out_specs=pl.BlockSpec((BLK, H), lambda i, be, bv: (i, 0)),
    )
    return pl.pallas_call(
        _down_body,
        grid_spec=grid_spec,
        out_shape=jax.ShapeDtypeStruct((NSLOT, H), jnp.float32),
    )(blke, blkv, mid, wd, ad, bd, sw3)


def _combine_body(pos_hbm, op_hbm, out_hbm, i0, i1, r0, r1, sem0, sem1):
    wid = lax.axis_index("s") * 2 + lax.axis_index("c")
    for c in range(CPW // 16):
        tb = wid * CPW + c * 16
        pltpu.sync_copy(pos_hbm.at[pl.ds(tb, 16)], i0)
        pltpu.sync_copy(pos_hbm.at[pl.ds(T + tb, 16)], i1)
        pltpu.async_copy(op_hbm.at[i0], r0, sem0).wait()
        pltpu.async_copy(op_hbm.at[i1], r1, sem1).wait()
        for row in range(16):
            def add_row(j, _, row=row):
                cc = j * 16
                r0[row, pl.ds(cc, 16)] = (r0[row, pl.ds(cc, 16)]
                                          + r1[row, pl.ds(cc, 16)])
                return 0
            lax.fori_loop(0, H // 16, add_row, 0)
        pltpu.sync_copy(r0, out_hbm.at[pl.ds(tb, 16)])


def _combine(pos, outp):
    mesh = plsc.VectorSubcoreMesh(core_axis_name="c", subcore_axis_name="s",
                                  num_cores=2, num_subcores=NTILES)
    f = pl.kernel(
        _combine_body,
        out_type=jax.ShapeDtypeStruct((T, H), jnp.float32),
        mesh=mesh,
        scratch_types=[
            pltpu.VMEM((16,), jnp.int32),
            pltpu.VMEM((16,), jnp.int32),
            pltpu.VMEM((16, H), jnp.float32),
            pltpu.VMEM((16, H), jnp.float32),
            pltpu.SemaphoreType.DMA,
            pltpu.SemaphoreType.DMA,
        ],
    )
    return f(pos, outp)


def kernel(hidden_states, router_w, base_gate_w, base_up_w, base_down_w,
           lora_gate_a, lora_gate_b, lora_up_a, lora_up_b,
           lora_down_a, lora_down_b):
    bf16 = jnp.bfloat16
    x = hidden_states.reshape(T, H)
    x_bf = x.astype(bf16)
    rw_pad = jnp.zeros((EPAD, H), jnp.float32).at[:E].set(router_w)
    logits_full, e01_full, w0_full = _router(x, rw_pad)
    router_logits = logits_full[:, :E]
    baseg, baseu = _base(x, base_gate_w, base_up_w)
    iota16 = jnp.arange(16, dtype=jnp.int32)
    stok, sw, blke, blkv, pos, _ = _route(iota16, e01_full[:, 0], w0_full[:, 0])
    x_pack = lax.bitcast_convert_type(x_bf.reshape(T, H // 2, 2), jnp.int32)
    xg32, gg, gu = _gather(stok, x_pack, baseg, baseu)
    xg = lax.bitcast_convert_type(xg32, bf16).reshape(NSLOT, H)
    mid = _gu(blke, blkv, xg, gg, gu,
              lora_gate_a.astype(bf16), lora_gate_b.astype(bf16),
              lora_up_a.astype(bf16), lora_up_b.astype(bf16))
    outp = _down(blke, blkv, mid, base_down_w, lora_down_a, lora_down_b,
                 sw.reshape(NB, 1, BLK))
    final = _combine(pos, outp)
    return final.reshape(1, T, H), router_logits


# speedup vs baseline: 1.4645x; 1.4299x over previous
import jax
import jax.numpy as jnp
from jax import lax
from jax.experimental import pallas as pl
from jax.experimental.pallas import tpu as pltpu
from jax.experimental.pallas import tpu_sc as plsc

T = 2048
H = 2048
I_DIM = 1024
E = 8
R = 602
EPAD = 128
BLK = 256
LOG2_BLK = 8
NB = 24
NSLOT = NB * BLK
NTILES = 16
TPW = T // NTILES
SPW = NSLOT // 32
CPW = T // 32
GCH = 8


def _router_body(x_ref, rw_ref, logits_ref, e01_ref, w0_ref):
    xb = x_ref[...]
    rw = rw_ref[...]
    lt = lax.dot_general(xb, rw, (((1,), (1,)), ((), ())),
                         preferred_element_type=jnp.float32)
    logits_ref[...] = lt
    lane = lax.broadcasted_iota(jnp.int32, lt.shape, 1)
    neg = jnp.float32(-jnp.inf)
    lm = jnp.where(lane < E, lt, neg)
    m0 = jnp.max(lm, axis=1, keepdims=True)
    e0 = jnp.min(jnp.where(lm == m0, lane, EPAD), axis=1, keepdims=True)
    lm1 = jnp.where(lane == e0, neg, lm)
    m1 = jnp.max(lm1, axis=1, keepdims=True)
    e1 = jnp.min(jnp.where(lm1 == m1, lane, EPAD), axis=1, keepdims=True)
    w0 = 1.0 / (1.0 + jnp.exp(m1 - m0))
    e01_ref[...] = jnp.broadcast_to(e0 + 16 * e1, lt.shape)
    w0_ref[...] = jnp.broadcast_to(w0, lt.shape)


def _router(x, rw_pad):
    return pl.pallas_call(
        _router_body,
        grid=(T // 256,),
        in_specs=[
            pl.BlockSpec((256, H), lambda i: (i, 0)),
            pl.BlockSpec((EPAD, H), lambda i: (0, 0)),
        ],
        out_specs=[
            pl.BlockSpec((256, EPAD), lambda i: (i, 0)),
            pl.BlockSpec((256, EPAD), lambda i: (i, 0)),
            pl.BlockSpec((256, EPAD), lambda i: (i, 0)),
        ],
        out_shape=[
            jax.ShapeDtypeStruct((T, EPAD), jnp.float32),
            jax.ShapeDtypeStruct((T, EPAD), jnp.int32),
            jax.ShapeDtypeStruct((T, EPAD), jnp.float32),
        ],
    )(x, rw_pad)


def _base_body(x_ref, wg_ref, wu_ref, bg_ref, bu_ref):
    xb = x_ref[...]
    cdim = (((1,), (1,)), ((), ()))
    bg_ref[...] = lax.dot_general(xb, wg_ref[...], cdim,
                                  preferred_element_type=jnp.float32)
    bu_ref[...] = lax.dot_general(xb, wu_ref[...], cdim,
                                  preferred_element_type=jnp.float32)


def _base(x, wg, wu):
    return pl.pallas_call(
        _base_body,
        grid=(T // 256,),
        in_specs=[
            pl.BlockSpec((256, H), lambda i: (i, 0)),
            pl.BlockSpec((I_DIM, H), lambda i: (0, 0)),
            pl.BlockSpec((I_DIM, H), lambda i: (0, 0)),
        ],
        out_specs=[
            pl.BlockSpec((256, I_DIM), lambda i: (i, 0)),
            pl.BlockSpec((256, I_DIM), lambda i: (i, 0)),
        ],
        out_shape=[
            jax.ShapeDtypeStruct((T, I_DIM), jnp.float32),
            jax.ShapeDtypeStruct((T, I_DIM), jnp.float32),
        ],
    )(x, wg, wu)


def _splat(v, p):
    return jnp.take(v, jnp.full((16,), p, jnp.int32))


def _route_body(iota_hbm, e01_hbm, w0_hbm,
                stok_hbm, sw_hbm, blke_hbm, blkv_hbm, pos_hbm, cnt_hbm,
                iov, e01v, w0v, ev, w1v, cnt, allcnt, zeri, zerf,
                tok, pos0, pos1, bvm, bvv):
    w = lax.axis_index("s")
    i32 = jnp.int32
    pltpu.sync_copy(iota_hbm, iov)
    iot = iov[...]
    pltpu.sync_copy(e01_hbm.at[pl.ds(w * TPW, TPW)], e01v)
    pltpu.sync_copy(w0_hbm.at[pl.ds(w * TPW, TPW)], w0v)
    cntv = jnp.zeros((16,), i32)
    for g in range(TPW // 16):
        pk = e01v[pl.ds(g * 16, 16)]
        e0 = pk & 15
        e1 = pk >> 4
        ev[pl.ds(g * 16, 16)] = e0
        ev[pl.ds(TPW + g * 16, 16)] = e1
        w1v[pl.ds(g * 16, 16)] = 1.0 - w0v[pl.ds(g * 16, 16)]
        tok[pl.ds(g * 16, 16)] = w * TPW + g * 16 + iot
        for p in range(16):
            cntv = cntv + jnp.where(iot == jnp.take(e0, e0 * 0 + p), 1, 0)
            cntv = cntv + jnp.where(iot == jnp.take(e1, e1 * 0 + p), 1, 0)
    cnt[...] = cntv
    nz = NSLOT // NTILES
    for j in range(nz // 16):
        zeri[pl.ds(j * 16, 16)] = jnp.zeros((16,), i32)
        zerf[pl.ds(j * 16, 16)] = jnp.zeros((16,), jnp.float32)
    pltpu.sync_copy(zeri, stok_hbm.at[pl.ds(w * nz, nz)])
    pltpu.sync_copy(zerf, sw_hbm.at[pl.ds(w * nz, nz)])
    pltpu.sync_copy(cnt, cnt_hbm.at[pl.ds(w * 16, 16)])
    plsc.subcore_barrier()
    pltpu.sync_copy(cnt_hbm, allcnt)
    tot = jnp.zeros((16,), i32)
    pre = jnp.zeros((16,), i32)
    for j in range(NTILES):
        row = allcnt[pl.ds(j * 16, 16)]
        tot = tot + row
        pre = pre + row * (j < w).astype(i32)
    pc = ((tot + (BLK - 1)) >> LOG2_BLK) << LOG2_BLK
    basev = jnp.zeros((16,), i32)
    accv = pc * 0
    nbc = []
    for e in range(E):
        basev = basev + jnp.where(iot == e, accv, 0)
        accv = accv + jnp.take(pc, pc * 0 + e)
        nbc.append(accv >> LOG2_BLK)
    start = basev + pre

    @pl.when(w == 0)
    def _write_block_tables():
        for half in range(2):
            b = iot + half * 16
            be = jnp.zeros((16,), i32)
            for e in range(E - 1):
                be = be + jnp.where(b >= nbc[e], 1, 0)
            valid = jnp.where(b < nbc[E - 1], 1, 0)
            be = be * valid + (E - 1) * (1 - valid)
            bvm[pl.ds(half * 16, 16)] = be
            bvv[pl.ds(half * 16, 16)] = valid
        pltpu.sync_copy(bvm, blke_hbm)
        pltpu.sync_copy(bvv, blkv_hbm)

    off = start
    gt = [jnp.where(iot > p, 1, 0) for p in range(16)]
    for k in range(2):
        dst = pos0 if k == 0 else pos1
        for g in range(TPW // 16):
            gbase = k * TPW + g * 16
            ids = ev[pl.ds(gbase, 16)]
            rank = jnp.zeros((16,), i32)
            hist = jnp.zeros((16,), i32)
            for p in range(16):
                sp = jnp.take(ids, ids * 0 + p)
                rank = rank + jnp.where(ids == sp, gt[p], 0)
                hist = hist + jnp.where(iot == sp, 1, 0)
            dst[pl.ds(g * 16, 16)] = jnp.take(off, ids) + rank
            off = off + hist
    pltpu.sync_copy(tok, stok_hbm.at[pos0])
    pltpu.sync_copy(tok, stok_hbm.at[pos1])
    pltpu.sync_copy(w0v, sw_hbm.at[pos0])
    pltpu.sync_copy(w1v, sw_hbm.at[pos1])
    pltpu.sync_copy(pos0, pos_hbm.at[pl.ds(w * TPW, TPW)])
    pltpu.sync_copy(pos1, pos_hbm.at[pl.ds(T + w * TPW, TPW)])


def _route(iota16, e01, w0):
    mesh = plsc.VectorSubcoreMesh(core_axis_name="c", subcore_axis_name="s",
                                  num_cores=1, num_subcores=NTILES)
    f = pl.kernel(
        _route_body,
        out_type=[
            jax.ShapeDtypeStruct((NSLOT,), jnp.int32),
            jax.ShapeDtypeStruct((NSLOT,), jnp.float32),
            jax.ShapeDtypeStruct((32,), jnp.int32),
            jax.ShapeDtypeStruct((32,), jnp.int32),
            jax.ShapeDtypeStruct((2 * T,), jnp.int32),
            jax.ShapeDtypeStruct((NTILES * 16,), jnp.int32),
        ],
        mesh=mesh,
        scratch_types=[
            pltpu.VMEM((16,), jnp.int32),
            pltpu.VMEM((TPW,), jnp.int32),
            pltpu.VMEM((TPW,), jnp.float32),
            pltpu.VMEM((2 * TPW,), jnp.int32),
            pltpu.VMEM((TPW,), jnp.float32),
            pltpu.VMEM((16,), jnp.int32),
            pltpu.VMEM((NTILES * 16,), jnp.int32),
            pltpu.VMEM((NSLOT // NTILES,), jnp.int32),
            pltpu.VMEM((NSLOT // NTILES,), jnp.float32),
            pltpu.VMEM((TPW,), jnp.int32),
            pltpu.VMEM((TPW,), jnp.int32),
            pltpu.VMEM((TPW,), jnp.int32),
            pltpu.VMEM((32,), jnp.int32),
            pltpu.VMEM((32,), jnp.int32),
        ],
    )
    return f(iota16, e01, w0)


def _gather_body(stok_hbm, x_hbm, bg_hbm, bu_hbm, xg_hbm, gg_hbm, gu_hbm,
                 idx0, idx1, x0, g0, u0, x1, g1, u1, sem0, sem1):
    wid = lax.axis_index("s") * 2 + lax.axis_index("c")
    base = wid * SPW
    nc = SPW // GCH
    bufs = [(idx0, x0, g0, u0, sem0), (idx1, x1, g1, u1, sem1)]

    def issue(c, bi):
        idxr, xr, gr, ur, sem = bufs[bi]
        b = base + c * GCH
        pltpu.sync_copy(stok_hbm.at[pl.ds(b, GCH)], idxr)
        return (pltpu.async_copy(x_hbm.at[idxr], xr, sem),
                pltpu.async_copy(bg_hbm.at[idxr], gr, sem),
                pltpu.async_copy(bu_hbm.at[idxr], ur, sem))

    handles = [issue(0, 0), None]
    for c in range(nc):
        bi = c % 2
        if c + 1 < nc:
            handles[1 - bi] = issue(c + 1, 1 - bi)
        for h in handles[bi]:
            h.wait()
        _, xr, gr, ur, _ = bufs[bi]
        b = base + c * GCH
        pltpu.sync_copy(xr, xg_hbm.at[pl.ds(b, GCH)])
        pltpu.sync_copy(gr, gg_hbm.at[pl.ds(b, GCH)])
        pltpu.sync_copy(ur, gu_hbm.at[pl.ds(b, GCH)])


def _gather(stok, x_bf, baseg, baseu):
    mesh = plsc.VectorSubcoreMesh(core_axis_name="c", subcore_axis_name="s",
                                  num_cores=2, num_subcores=NTILES)
    f = pl.kernel(
        _gather_body,
        out_type=[
            jax.ShapeDtypeStruct((NSLOT, H), jnp.float32),
            jax.ShapeDtypeStruct((NSLOT, I_DIM), jnp.float32),
            jax.ShapeDtypeStruct((NSLOT, I_DIM), jnp.float32),
        ],
        mesh=mesh,
        scratch_types=[
            pltpu.VMEM((GCH,), jnp.int32),
            pltpu.VMEM((GCH,), jnp.int32),
            pltpu.VMEM((GCH, H), jnp.float32),
            pltpu.VMEM((GCH, I_DIM), jnp.float32),
            pltpu.VMEM((GCH, I_DIM), jnp.float32),
            pltpu.VMEM((GCH, H), jnp.float32),
            pltpu.VMEM((GCH, I_DIM), jnp.float32),
            pltpu.VMEM((GCH, I_DIM), jnp.float32),
            pltpu.SemaphoreType.DMA,
            pltpu.SemaphoreType.DMA,
        ],
    )
    return f(stok, x_bf, baseg, baseu)


def _gu_body(be_ref, bv_ref, xg_ref, gg_ref, gu_ref,
             ag_ref, bg_ref, au_ref, bu_ref, mid_ref):
    i = pl.program_id(0)

    @pl.when(bv_ref[i] == 1)
    def _():
        xb = xg_ref[...]
        cdim = (((1,), (1,)), ((), ()))
        f32 = jnp.float32
        g = gg_ref[...] + lax.dot_general(
            lax.dot_general(xb, ag_ref[0], cdim, preferred_element_type=f32),
            bg_ref[0], cdim, preferred_element_type=f32)
        u = gu_ref[...] + lax.dot_general(
            lax.dot_general(xb, au_ref[0], cdim, preferred_element_type=f32),
            bu_ref[0], cdim, preferred_element_type=f32)
        mid_ref[...] = g * jax.nn.sigmoid(g) * u


def _gu(blke, blkv, xg, gg, gu, ag, bg, au, bu):
    grid_spec = pltpu.PrefetchScalarGridSpec(
        num_scalar_prefetch=2,
        grid=(NB,),
        in_specs=[
            pl.BlockSpec((BLK, H), lambda i, be, bv: (i, 0)),
            pl.BlockSpec((BLK, I_DIM), lambda i, be, bv: (i, 0)),
            pl.BlockSpec((BLK, I_DIM), lambda i, be, bv: (i, 0)),
            pl.BlockSpec((1, R, H), lambda i, be, bv: (be[i], 0, 0)),
            pl.BlockSpec((1, I_DIM, R), lambda i, be, bv: (be[i], 0, 0)),
            pl.BlockSpec((1, R, H), lambda i, be, bv: (be[i], 0, 0)),
            pl.BlockSpec((1, I_DIM, R), lambda i, be, bv: (be[i], 0, 0)),
        ],
        out_specs=pl.BlockSpec((BLK, I_DIM), lambda i, be, bv: (i, 0)),
    )
    return pl.pallas_call(
        _gu_body,
        grid_spec=grid_spec,
        out_shape=jax.ShapeDtypeStruct((NSLOT, I_DIM), jnp.float32),
    )(blke, blkv, xg, gg, gu, ag, bg, au, bu)


def _down_body(be_ref, bv_ref, mid_ref, wd_ref, ad_ref, bd_ref, sw_ref, out_ref):
    i = pl.program_id(0)

    @pl.when(bv_ref[i] == 1)
    def _():
        mid = mid_ref[...]
        cdim = (((1,), (1,)), ((), ()))
        f32 = jnp.float32
        d = (lax.dot_general(mid, wd_ref[...], cdim, preferred_element_type=f32)
             + lax.dot_general(
                 lax.dot_general(mid, ad_ref[0], cdim, preferred_element_type=f32),
                 bd_ref[0], cdim, preferred_element_type=f32))
        out_ref[...] = d * sw_ref[0, 0, :][:, None]


def _down(blke, blkv, mid, wd, ad, bd, sw3):
    grid_spec = pltpu.PrefetchScalarGridSpec(
        num_scalar_prefetch=2,
        grid=(NB,),
        in_specs=[
            pl.BlockSpec((BLK, I_DIM), lambda i, be, bv: (i, 0)),
            pl.BlockSpec((H, I_DIM), lambda i, be, bv: (0, 0)),
            pl.BlockSpec((1, R, I_DIM), lambda i, be, bv: (be[i], 0, 0)),
            pl.BlockSpec((1, H, R), lambda i, be, bv: (be[i], 0, 0)),
            pl.BlockSpec((1, 1, BLK), lambda i, be, bv: (i, 0, 0)),
        ],
        out_specs=pl.BlockSpec((BLK, H), lambda i, be, bv: (i, 0)),
    )
    return pl.pallas_call(
        _down_body,
        grid_spec=grid_spec,
        out_shape=jax.ShapeDtypeStruct((NSLOT, H), jnp.float32),
    )(blke, blkv, mid, wd, ad, bd, sw3)


def _combine_body(pos_hbm, op_hbm, out_hbm, i0, i1, r0, r1, sem0, sem1):
    wid = lax.axis_index("s") * 2 + lax.axis_index("c")
    for c in range(CPW // 16):
        tb = wid * CPW + c * 16
        pltpu.sync_copy(pos_hbm.at[pl.ds(tb, 16)], i0)
        pltpu.sync_copy(pos_hbm.at[pl.ds(T + tb, 16)], i1)
        pltpu.async_copy(op_hbm.at[i0], r0, sem0).wait()
        pltpu.async_copy(op_hbm.at[i1], r1, sem1).wait()
        for row in range(16):
            def add_row(j, _, row=row):
                cc = j * 16
                r0[row, pl.ds(cc, 16)] = (r0[row, pl.ds(cc, 16)]
                                          + r1[row, pl.ds(cc, 16)])
                return 0
            lax.fori_loop(0, H // 16, add_row, 0)
        pltpu.sync_copy(r0, out_hbm.at[pl.ds(tb, 16)])


def _combine(pos, outp):
    mesh = plsc.VectorSubcoreMesh(core_axis_name="c", subcore_axis_name="s",
                                  num_cores=2, num_subcores=NTILES)
    f = pl.kernel(
        _combine_body,
        out_type=jax.ShapeDtypeStruct((T, H), jnp.float32),
        mesh=mesh,
        scratch_types=[
            pltpu.VMEM((16,), jnp.int32),
            pltpu.VMEM((16,), jnp.int32),
            pltpu.VMEM((16, H), jnp.float32),
            pltpu.VMEM((16, H), jnp.float32),
            pltpu.SemaphoreType.DMA,
            pltpu.SemaphoreType.DMA,
        ],
    )
    return f(pos, outp)


def kernel(hidden_states, router_w, base_gate_w, base_up_w, base_down_w,
           lora_gate_a, lora_gate_b, lora_up_a, lora_up_b,
           lora_down_a, lora_down_b):
    bf16 = jnp.bfloat16
    x = hidden_states.reshape(T, H)
    rw_pad = jnp.zeros((EPAD, H), jnp.float32).at[:E].set(router_w)
    logits_full, e01_full, w0_full = _router(x, rw_pad)
    router_logits = logits_full[:, :E]
    baseg, baseu = _base(x, base_gate_w, base_up_w)
    iota16 = jnp.arange(16, dtype=jnp.int32)
    stok, sw, blke, blkv, pos, _ = _route(iota16, e01_full[:, 0], w0_full[:, 0])
    xg, gg, gu = _gather(stok, x, baseg, baseu)
    mid = _gu(blke, blkv, xg.astype(bf16), gg, gu,
              lora_gate_a.astype(bf16), lora_gate_b.astype(bf16),
              lora_up_a.astype(bf16), lora_up_b.astype(bf16))
    outp = _down(blke, blkv, mid, base_down_w, lora_down_a, lora_down_b,
                 sw.reshape(NB, 1, BLK))
    final = _combine(pos, outp)
    return final.reshape(1, T, H), router_logits


# speedup vs baseline: 1.5501x; 1.0585x over previous
import jax
import jax.numpy as jnp
from jax import lax
from jax.experimental import pallas as pl
from jax.experimental.pallas import tpu as pltpu
from jax.experimental.pallas import tpu_sc as plsc

T = 2048
H = 2048
I_DIM = 1024
E = 8
R = 602
EPAD = 128
BLK = 256
LOG2_BLK = 8
NB = 24
NSLOT = NB * BLK
NTILES = 16
TPW = T // NTILES
SPW = NSLOT // 32
CPW = T // 32
GCH = 16


def _router_body(x_ref, rw_ref, logits_ref, e01_ref, w0_ref):
    xb = x_ref[...]
    rw = rw_ref[...]
    lt = lax.dot_general(xb, rw, (((1,), (1,)), ((), ())),
                         preferred_element_type=jnp.float32)
    logits_ref[...] = lt
    lane = lax.broadcasted_iota(jnp.int32, lt.shape, 1)
    neg = jnp.float32(-jnp.inf)
    lm = jnp.where(lane < E, lt, neg)
    m0 = jnp.max(lm, axis=1, keepdims=True)
    e0 = jnp.min(jnp.where(lm == m0, lane, EPAD), axis=1, keepdims=True)
    lm1 = jnp.where(lane == e0, neg, lm)
    m1 = jnp.max(lm1, axis=1, keepdims=True)
    e1 = jnp.min(jnp.where(lm1 == m1, lane, EPAD), axis=1, keepdims=True)
    w0 = 1.0 / (1.0 + jnp.exp(m1 - m0))
    e01_ref[...] = jnp.broadcast_to(e0 + 16 * e1, lt.shape)
    w0_ref[...] = jnp.broadcast_to(w0, lt.shape)


def _router(x, rw_pad):
    return pl.pallas_call(
        _router_body,
        grid=(T // 256,),
        in_specs=[
            pl.BlockSpec((256, H), lambda i: (i, 0)),
            pl.BlockSpec((EPAD, H), lambda i: (0, 0)),
        ],
        out_specs=[
            pl.BlockSpec((256, EPAD), lambda i: (i, 0)),
            pl.BlockSpec((256, EPAD), lambda i: (i, 0)),
            pl.BlockSpec((256, EPAD), lambda i: (i, 0)),
        ],
        out_shape=[
            jax.ShapeDtypeStruct((T, EPAD), jnp.float32),
            jax.ShapeDtypeStruct((T, EPAD), jnp.int32),
            jax.ShapeDtypeStruct((T, EPAD), jnp.float32),
        ],
    )(x, rw_pad)


def _base_body(x_ref, wg_ref, wu_ref, bg_ref, bu_ref):
    xb = x_ref[...]
    cdim = (((1,), (1,)), ((), ()))
    bg_ref[...] = lax.dot_general(xb, wg_ref[...], cdim,
                                  preferred_element_type=jnp.float32)
    bu_ref[...] = lax.dot_general(xb, wu_ref[...], cdim,
                                  preferred_element_type=jnp.float32)


def _base(x, wg, wu):
    return pl.pallas_call(
        _base_body,
        grid=(T // 256,),
        in_specs=[
            pl.BlockSpec((256, H), lambda i: (i, 0)),
            pl.BlockSpec((I_DIM, H), lambda i: (0, 0)),
            pl.BlockSpec((I_DIM, H), lambda i: (0, 0)),
        ],
        out_specs=[
            pl.BlockSpec((256, I_DIM), lambda i: (i, 0)),
            pl.BlockSpec((256, I_DIM), lambda i: (i, 0)),
        ],
        out_shape=[
            jax.ShapeDtypeStruct((T, I_DIM), jnp.float32),
            jax.ShapeDtypeStruct((T, I_DIM), jnp.float32),
        ],
    )(x, wg, wu)


def _splat(v, p):
    return jnp.take(v, jnp.full((16,), p, jnp.int32))


def _route_body(iota_hbm, e01_hbm, w0_hbm,
                stok_hbm, sw_hbm, blke_hbm, blkv_hbm, pos_hbm, cnt_hbm,
                iov, e01v, w0v, ev, w1v, cnt, allcnt, zeri, zerf,
                tok, pos0, pos1, bvm, bvv):
    w = lax.axis_index("s")
    i32 = jnp.int32
    pltpu.sync_copy(iota_hbm, iov)
    iot = iov[...]
    pltpu.sync_copy(e01_hbm.at[pl.ds(w * TPW, TPW)], e01v)
    pltpu.sync_copy(w0_hbm.at[pl.ds(w * TPW, TPW)], w0v)
    cntv = jnp.zeros((16,), i32)
    for g in range(TPW // 16):
        pk = e01v[pl.ds(g * 16, 16)]
        e0 = pk & 15
        e1 = pk >> 4
        ev[pl.ds(g * 16, 16)] = e0
        ev[pl.ds(TPW + g * 16, 16)] = e1
        w1v[pl.ds(g * 16, 16)] = 1.0 - w0v[pl.ds(g * 16, 16)]
        tok[pl.ds(g * 16, 16)] = w * TPW + g * 16 + iot
        for p in range(16):
            cntv = cntv + jnp.where(iot == jnp.take(e0, e0 * 0 + p), 1, 0)
            cntv = cntv + jnp.where(iot == jnp.take(e1, e1 * 0 + p), 1, 0)
    cnt[...] = cntv
    nz = NSLOT // NTILES
    for j in range(nz // 16):
        zeri[pl.ds(j * 16, 16)] = jnp.zeros((16,), i32)
        zerf[pl.ds(j * 16, 16)] = jnp.zeros((16,), jnp.float32)
    pltpu.sync_copy(zeri, stok_hbm.at[pl.ds(w * nz, nz)])
    pltpu.sync_copy(zerf, sw_hbm.at[pl.ds(w * nz, nz)])
    pltpu.sync_copy(cnt, cnt_hbm.at[pl.ds(w * 16, 16)])
    plsc.subcore_barrier()
    pltpu.sync_copy(cnt_hbm, allcnt)
    tot = jnp.zeros((16,), i32)
    pre = jnp.zeros((16,), i32)
    for j in range(NTILES):
        row = allcnt[pl.ds(j * 16, 16)]
        tot = tot + row
        pre = pre + row * (j < w).astype(i32)
    pc = ((tot + (BLK - 1)) >> LOG2_BLK) << LOG2_BLK
    basev = jnp.zeros((16,), i32)
    accv = pc * 0
    nbc = []
    for e in range(E):
        basev = basev + jnp.where(iot == e, accv, 0)
        accv = accv + jnp.take(pc, pc * 0 + e)
        nbc.append(accv >> LOG2_BLK)
    start = basev + pre

    @pl.when(w == 0)
    def _write_block_tables():
        for half in range(2):
            b = iot + half * 16
            be = jnp.zeros((16,), i32)
            for e in range(E - 1):
                be = be + jnp.where(b >= nbc[e], 1, 0)
            valid = jnp.where(b < nbc[E - 1], 1, 0)
            be = be * valid + (E - 1) * (1 - valid)
            bvm[pl.ds(half * 16, 16)] = be
            bvv[pl.ds(half * 16, 16)] = valid
        pltpu.sync_copy(bvm, blke_hbm)
        pltpu.sync_copy(bvv, blkv_hbm)

    off = start
    gt = [jnp.where(iot > p, 1, 0) for p in range(16)]
    for k in range(2):
        dst = pos0 if k == 0 else pos1
        for g in range(TPW // 16):
            gbase = k * TPW + g * 16
            ids = ev[pl.ds(gbase, 16)]
            rank = jnp.zeros((16,), i32)
            hist = jnp.zeros((16,), i32)
            for p in range(16):
                sp = jnp.take(ids, ids * 0 + p)
                rank = rank + jnp.where(ids == sp, gt[p], 0)
                hist = hist + jnp.where(iot == sp, 1, 0)
            dst[pl.ds(g * 16, 16)] = jnp.take(off, ids) + rank
            off = off + hist
    pltpu.sync_copy(tok, stok_hbm.at[pos0])
    pltpu.sync_copy(tok, stok_hbm.at[pos1])
    pltpu.sync_copy(w0v, sw_hbm.at[pos0])
    pltpu.sync_copy(w1v, sw_hbm.at[pos1])
    pltpu.sync_copy(pos0, pos_hbm.at[pl.ds(w * TPW, TPW)])
    pltpu.sync_copy(pos1, pos_hbm.at[pl.ds(T + w * TPW, TPW)])


def _route(iota16, e01, w0):
    mesh = plsc.VectorSubcoreMesh(core_axis_name="c", subcore_axis_name="s",
                                  num_cores=1, num_subcores=NTILES)
    f = pl.kernel(
        _route_body,
        out_type=[
            jax.ShapeDtypeStruct((NSLOT,), jnp.int32),
            jax.ShapeDtypeStruct((NSLOT,), jnp.float32),
            jax.ShapeDtypeStruct((32,), jnp.int32),
            jax.ShapeDtypeStruct((32,), jnp.int32),
            jax.ShapeDtypeStruct((2 * T,), jnp.int32),
            jax.ShapeDtypeStruct((NTILES * 16,), jnp.int32),
        ],
        mesh=mesh,
        scratch_types=[
            pltpu.VMEM((16,), jnp.int32),
            pltpu.VMEM((TPW,), jnp.int32),
            pltpu.VMEM((TPW,), jnp.float32),
            pltpu.VMEM((2 * TPW,), jnp.int32),
            pltpu.VMEM((TPW,), jnp.float32),
            pltpu.VMEM((16,), jnp.int32),
            pltpu.VMEM((NTILES * 16,), jnp.int32),
            pltpu.VMEM((NSLOT // NTILES,), jnp.int32),
            pltpu.VMEM((NSLOT // NTILES,), jnp.float32),
            pltpu.VMEM((TPW,), jnp.int32),
            pltpu.VMEM((TPW,), jnp.int32),
            pltpu.VMEM((TPW,), jnp.int32),
            pltpu.VMEM((32,), jnp.int32),
            pltpu.VMEM((32,), jnp.int32),
        ],
    )
    return f(iota16, e01, w0)


def _gather_body(stok_hbm, x_hbm, xg_hbm,
                 idx0, idx1, x0, x1, sem0, sem1):
    wid = lax.axis_index("s") * 2 + lax.axis_index("c")
    base = wid * SPW
    nc = SPW // GCH
    bufs = [(idx0, x0, sem0), (idx1, x1, sem1)]

    def issue(c, bi):
        idxr, xr, sem = bufs[bi]
        b = base + c * GCH
        pltpu.sync_copy(stok_hbm.at[pl.ds(b, GCH)], idxr)
        return pltpu.async_copy(x_hbm.at[idxr], xr, sem)

    handles = [issue(0, 0), None]
    for c in range(nc):
        bi = c % 2
        if c + 1 < nc:
            handles[1 - bi] = issue(c + 1, 1 - bi)
        handles[bi].wait()
        _, xr, _ = bufs[bi]
        b = base + c * GCH
        pltpu.sync_copy(xr, xg_hbm.at[pl.ds(b, GCH)])


def _gather(stok, x):
    mesh = plsc.VectorSubcoreMesh(core_axis_name="c", subcore_axis_name="s",
                                  num_cores=2, num_subcores=NTILES)
    f = pl.kernel(
        _gather_body,
        out_type=jax.ShapeDtypeStruct((NSLOT, H), jnp.float32),
        mesh=mesh,
        scratch_types=[
            pltpu.VMEM((GCH,), jnp.int32),
            pltpu.VMEM((GCH,), jnp.int32),
            pltpu.VMEM((GCH, H), jnp.float32),
            pltpu.VMEM((GCH, H), jnp.float32),
            pltpu.SemaphoreType.DMA,
            pltpu.SemaphoreType.DMA,
        ],
    )
    return f(stok, x)


def _gu_body(be_ref, bv_ref, xg_ref, wg_ref, wu_ref,
             ag_ref, bg_ref, au_ref, bu_ref, mid_ref):
    i = pl.program_id(0)

    @pl.when(bv_ref[i] == 1)
    def _():
        xb = xg_ref[...]
        cdim = (((1,), (1,)), ((), ()))
        f32 = jnp.float32
        g = (lax.dot_general(xb, wg_ref[...], cdim, preferred_element_type=f32)
             + lax.dot_general(
                 lax.dot_general(xb, ag_ref[0], cdim, preferred_element_type=f32),
                 bg_ref[0], cdim, preferred_element_type=f32))
        u = (lax.dot_general(xb, wu_ref[...], cdim, preferred_element_type=f32)
             + lax.dot_general(
                 lax.dot_general(xb, au_ref[0], cdim, preferred_element_type=f32),
                 bu_ref[0], cdim, preferred_element_type=f32))
        mid_ref[...] = g * jax.nn.sigmoid(g) * u


def _gu(blke, blkv, xg, wg, wu, ag, bg, au, bu):
    grid_spec = pltpu.PrefetchScalarGridSpec(
        num_scalar_prefetch=2,
        grid=(NB,),
        in_specs=[
            pl.BlockSpec((BLK, H), lambda i, be, bv: (i, 0)),
            pl.BlockSpec((I_DIM, H), lambda i, be, bv: (0, 0)),
            pl.BlockSpec((I_DIM, H), lambda i, be, bv: (0, 0)),
            pl.BlockSpec((1, R, H), lambda i, be, bv: (be[i], 0, 0)),
            pl.BlockSpec((1, I_DIM, R), lambda i, be, bv: (be[i], 0, 0)),
            pl.BlockSpec((1, R, H), lambda i, be, bv: (be[i], 0, 0)),
            pl.BlockSpec((1, I_DIM, R), lambda i, be, bv: (be[i], 0, 0)),
        ],
        out_specs=pl.BlockSpec((BLK, I_DIM), lambda i, be, bv: (i, 0)),
    )
    return pl.pallas_call(
        _gu_body,
        grid_spec=grid_spec,
        out_shape=jax.ShapeDtypeStruct((NSLOT, I_DIM), jnp.float32),
    )(blke, blkv, xg, wg, wu, ag, bg, au, bu)


def _down_body(be_ref, bv_ref, mid_ref, wd_ref, ad_ref, bd_ref, sw_ref, out_ref):
    i = pl.program_id(0)

    @pl.when(bv_ref[i] == 1)
    def _():
        mid = mid_ref[...]
        cdim = (((1,), (1,)), ((), ()))
        f32 = jnp.float32
        d = (lax.dot_general(mid, wd_ref[...], cdim, preferred_element_type=f32)
             + lax.dot_general(
                 lax.dot_general(mid, ad_ref[0], cdim, preferred_element_type=f32),
                 bd_ref[0], cdim, preferred_element_type=f32))
        out_ref[...] = d * sw_ref[0, 0, :][:, None]


def _down(blke, blkv, mid, wd, ad, bd, sw3):
    grid_spec = pltpu.PrefetchScalarGridSpec(
        num_scalar_prefetch=2,
        grid=(NB,),
        in_specs=[
            pl.BlockSpec((BLK, I_DIM), lambda i, be, bv: (i, 0)),
            pl.BlockSpec((H, I_DIM), lambda i, be, bv: (0, 0)),
            pl.BlockSpec((1, R, I_DIM), lambda i, be, bv: (be[i], 0, 0)),
            pl.BlockSpec((1, H, R), lambda i, be, bv: (be[i], 0, 0)),
            pl.BlockSpec((1, 1, BLK), lambda i, be, bv: (i, 0, 0)),
        ],
        out_specs=pl.BlockSpec((BLK, H), lambda i, be, bv: (i, 0)),
    )
    return pl.pallas_call(
        _down_body,
        grid_spec=grid_spec,
        out_shape=jax.ShapeDtypeStruct((NSLOT, H), jnp.float32),
    )(blke, blkv, mid, wd, ad, bd, sw3)


def _combine_body(pos_hbm, op_hbm, out_hbm, i0, i1, r0, r1, sem0, sem1):
    wid = lax.axis_index("s") * 2 + lax.axis_index("c")
    for c in range(CPW // 16):
        tb = wid * CPW + c * 16
        pltpu.sync_copy(pos_hbm.at[pl.ds(tb, 16)], i0)
        pltpu.sync_copy(pos_hbm.at[pl.ds(T + tb, 16)], i1)
        pltpu.async_copy(op_hbm.at[i0], r0, sem0).wait()
        pltpu.async_copy(op_hbm.at[i1], r1, sem1).wait()
        for row in range(16):
            def add_row(j, _, row=row):
                cc = j * 16
                r0[row, pl.ds(cc, 16)] = (r0[row, pl.ds(cc, 16)]
                                          + r1[row, pl.ds(cc, 16)])
                return 0
            lax.fori_loop(0, H // 16, add_row, 0)
        pltpu.sync_copy(r0, out_hbm.at[pl.ds(tb, 16)])


def _combine(pos, outp):
    mesh = plsc.VectorSubcoreMesh(core_axis_name="c", subcore_axis_name="s",
                                  num_cores=2, num_subcores=NTILES)
    f = pl.kernel(
        _combine_body,
        out_type=jax.ShapeDtypeStruct((T, H), jnp.float32),
        mesh=mesh,
        scratch_types=[
            pltpu.VMEM((16,), jnp.int32),
            pltpu.VMEM((16,), jnp.int32),
            pltpu.VMEM((16, H), jnp.float32),
            pltpu.VMEM((16, H), jnp.float32),
            pltpu.SemaphoreType.DMA,
            pltpu.SemaphoreType.DMA,
        ],
    )
    return f(pos, outp)


def kernel(hidden_states, router_w, base_gate_w, base_up_w, base_down_w,
           lora_gate_a, lora_gate_b, lora_up_a, lora_up_b,
           lora_down_a, lora_down_b):
    bf16 = jnp.bfloat16
    x = hidden_states.reshape(T, H)
    rw_pad = jnp.zeros((EPAD, H), jnp.float32).at[:E].set(router_w)
    logits_full, e01_full, w0_full = _router(x, rw_pad)
    router_logits = logits_full[:, :E]
    iota16 = jnp.arange(16, dtype=jnp.int32)
    stok, sw, blke, blkv, pos, _ = _route(iota16, e01_full[:, 0], w0_full[:, 0])
    xg = _gather(stok, x)
    mid = _gu(blke, blkv, xg.astype(bf16),
              base_gate_w.astype(bf16), base_up_w.astype(bf16),
              lora_gate_a.astype(bf16), lora_gate_b.astype(bf16),
              lora_up_a.astype(bf16), lora_up_b.astype(bf16))
    outp = _down(blke, blkv, mid, base_down_w, lora_down_a, lora_down_b,
                 sw.reshape(NB, 1, BLK))
    final = _combine(pos, outp)
    return final.reshape(1, T, H), router_logits
